# Initial kernel scaffold; baseline (speedup 1.0000x reference)
import jax
import jax.numpy as jnp
from jax.experimental import pallas as pl


def kernel(x, params, edge_index, batch):
    raise NotImplementedError("write your pallas kernel here")


# speedup vs baseline: 2.3411x; 2.3411x over previous
import functools

import jax
import jax.numpy as jnp
from jax import lax
from jax.experimental import pallas as pl
from jax.experimental.pallas import tpu as pltpu

N = 10000
E = 320000
DIM = 128
NG = 64
NH = 2
HD = DIM // NH
NS = 25
R = 1000
GRID = N // R
SCALE = 1.0 / (DIM ** 0.5)
F32 = jnp.float32


def _relu(v):
    return jnp.maximum(v, 0.0)


def _dot(a, b):
    return lax.dot_general(a, b, (((1,), (0,)), ((), ())),
                           preferred_element_type=F32)


def _dot3(a, w):
    return lax.dot_general(a, w, (((2,), (0,)), ((), ())),
                           preferred_element_type=F32)


_row = pl.BlockSpec((R, DIM), lambda i: (i, 0))
_row2 = pl.BlockSpec((R, 2 * DIM), lambda i: (i, 0))
_seg = pl.BlockSpec((2, R, DIM), lambda i: (0, i, 0))
_col = pl.BlockSpec((R, 1), lambda i: (i, 0))
_mat = pl.BlockSpec((DIM, DIM), lambda i: (0, 0))
_bia = pl.BlockSpec((1, DIM), lambda i: (0, 0))


def _mlp3_body(x, w1, b1, w2, b2, w3, b3, o):
    h = _relu(_dot(x[...], w1[...]) + b1[...])
    h = _relu(_dot(h, w2[...]) + b2[...])
    o[...] = _relu(_dot(h, w3[...]) + b3[...])


def _conv_lin_body(s, cnt, h, wl, bl, wr, w4, b4, o):
    agg = (s[0] + s[1]) / jnp.maximum(cnt[...], 1.0)
    hh = _relu(_dot(agg, wl[...]) + bl[...] + _dot(h[...], wr[...]))
    o[...] = _relu(_dot(hh, w4[...]) + b4[...])


def _conv_gcnprep_body(s, cnt, h, wl, bl, wr, w6, b6, wg, bg, wk, wv,
                       hkv_o, dinv_o):
    agg = (s[0] + s[1]) / jnp.maximum(cnt[...], 1.0)
    hh = _relu(_dot(agg, wl[...]) + bl[...] + _dot(h[...], wr[...]))
    h6 = _relu(_dot(hh, w6[...]) + b6[...])
    xg = _dot(h6, wg[...]) + bg[...]
    dinv = lax.rsqrt(cnt[...] + 1.0)
    hk = _dot(xg, wk[...]) * dinv
    hv = _dot(xg, wv[...]) * dinv
    hkv_o[...] = jnp.concatenate([hk, hv], axis=1)
    dinv_o[...] = dinv


def _pool1_body(sk, sv, hkv, dinv, batch, s1, wq, bq, bk, bv, den_o, num_o):
    i = pl.program_id(0)

    @pl.when(i == 0)
    def _():
        den_o[...] = jnp.zeros_like(den_o)
        num_o[...] = jnp.zeros_like(num_o)

    di = dinv[...]
    K = di * (sk[0] + sk[1] + hkv[:, :DIM]) + bk[...]
    V = di * (sv[0] + sv[1] + hkv[:, DIM:]) + bv[...]
    Qp = _dot(s1[...], wq[...]) + bq[...]
    onehot = (batch[...] == lax.broadcasted_iota(jnp.int32, (R, NG), 1)
              ).astype(F32)
    for h in range(NH):
        sl = slice(h * HD, (h + 1) * HD)
        Kh, Vh, Qh = K[:, sl], V[:, sl], Qp[:, sl]
        S = lax.dot_general(Kh, Qh, (((1,), (1,)), ((), ())),
                            preferred_element_type=F32) * SCALE
        w = jnp.exp(S)
        den_o[:, h * NS:(h + 1) * NS] += lax.dot_general(
            onehot, w, (((0,), (0,)), ((), ())), preferred_element_type=F32)
        for q in range(NS):
            ow = onehot * w[:, q:q + 1]
            num_o[q, :, sl] += lax.dot_general(
                ow, Vh, (((0,), (0,)), ((), ())), preferred_element_type=F32)


def _pooltail_body(num, den, s1, s3,
                   wq1, bq1, wo1, bo1,
                   wq2, bq2, wk2, bk2, wv2, bv2, wo2, bo2,
                   wq3, bq3, wk3, bk3, wv3, bv3, wo3, bo3,
                   wg2, bg2, out):
    Qp1 = _dot(s1[...], wq1[...]) + bq1[...]
    d = den[...]
    rows = []
    for q in range(NS):
        numq = num[q]
        d0 = d[:, q:q + 1]
        d1 = d[:, NS + q:NS + q + 1]
        v0 = jnp.where(d0 > 0, numq[:, :HD] / jnp.maximum(d0, 1e-30), 0.0)
        v1 = jnp.where(d1 > 0, numq[:, HD:] / jnp.maximum(d1, 1e-30), 0.0)
        o = jnp.concatenate([v0, v1], axis=1) + Qp1[q:q + 1, :]
        rows.append(o + _relu(_dot(o, wo1[...]) + bo1[...]))
    bx = jnp.stack(rows, axis=1)

    def lin3(a, w, b):
        return _dot3(a, w[...]) + b[...]

    def sab(bx2, Qf, wk, bk_, wv, bv_, wo, bo_):
        K2 = lin3(bx2, wk, bk_)
        V2 = lin3(bx2, wv, bv_)
        outs = []
        for h in range(NH):
            sl = slice(h * HD, (h + 1) * HD)
            sc = lax.dot_general(Qf[..., sl], K2[..., sl],
                                 (((2,), (2,)), ((0,), (0,))),
                                 preferred_element_type=F32) * SCALE
            m = jnp.max(sc, axis=-1, keepdims=True)
            e = jnp.exp(sc - m)
            A = e / jnp.sum(e, axis=-1, keepdims=True)
            outs.append(Qf[..., sl] + lax.dot_general(
                A, V2[..., sl], (((2,), (1,)), ((0,), (0,))),
                preferred_element_type=F32))
        o2 = jnp.concatenate(outs, axis=-1)
        return o2 + _relu(lin3(o2, wo, bo_))

    bx = sab(bx, lin3(bx, wq2, bq2), wk2, bk2, wv2, bv2, wo2, bo2)

    Qp3 = _dot(s3[...], wq3[...]) + bq3[...]
    K3 = lin3(bx, wk3, bk3)
    V3 = lin3(bx, wv3, bv3)
    outs = []
    for h in range(NH):
        sl = slice(h * HD, (h + 1) * HD)
        sc = jnp.sum(K3[..., sl] * Qp3[0:1, None, sl], axis=-1) * SCALE
        m = jnp.max(sc, axis=-1, keepdims=True)
        e = jnp.exp(sc - m)
        A = e / jnp.sum(e, axis=-1, keepdims=True)
        outs.append(Qp3[0:1, sl] + jnp.sum(A[..., None] * V3[..., sl], axis=1))
    o3 = jnp.concatenate(outs, axis=-1)
    bx3 = o3 + _relu(_dot(o3, wo3[...]) + bo3[...])
    out[...] = _dot(bx3, wg2[...]) + bg2[...]


def _segsum(tab, src, dst):
    s = jnp.zeros((N, tab.shape[-1]), tab.dtype).at[dst].add(tab[src])
    return jnp.stack([s, jnp.zeros_like(s)], axis=0)


def _counts(dst):
    return jnp.zeros((N,), F32).at[dst].add(1.0)


def kernel(x, params, edge_index, batch):
    p = params
    src, dst = edge_index[0], edge_index[1]

    def wt(nm):
        return p[nm + '_W'].T

    def wtk(key):
        return p[key].T

    def b2(nm):
        return p[nm + '_b'].reshape(1, DIM)

    def b2k(key):
        return p[key].reshape(1, DIM)

    cnt = _counts(dst).reshape(N, 1)
    batch2 = batch.reshape(N, 1)

    h3 = pl.pallas_call(
        _mlp3_body, grid=(GRID,),
        in_specs=[_row, _mat, _bia, _mat, _bia, _mat, _bia],
        out_specs=_row,
        out_shape=jax.ShapeDtypeStruct((N, DIM), F32),
    )(x, wt('lin1'), b2('lin1'), wt('lin2'), b2('lin2'), wt('lin3'), b2('lin3'))

    h = h3
    for conv, nxt in (('conv1', 'lin4'), ('conv2', 'lin5')):
        s = _segsum(h, src, dst)
        h = pl.pallas_call(
            _conv_lin_body, grid=(GRID,),
            in_specs=[_seg, _col, _row, _mat, _bia, _mat, _mat, _bia],
            out_specs=_row,
            out_shape=jax.ShapeDtypeStruct((N, DIM), F32),
        )(s, cnt, h, wtk(conv + '_Wl'), b2k(conv + '_bl'), wtk(conv + '_Wr'),
          wt(nxt), b2(nxt))

    s = _segsum(h, src, dst)
    hkv, dinv = pl.pallas_call(
        _conv_gcnprep_body, grid=(GRID,),
        in_specs=[_seg, _col, _row, _mat, _bia, _mat, _mat, _bia, _mat, _bia,
                  _mat, _mat],
        out_specs=[_row2, _col],
        out_shape=[jax.ShapeDtypeStruct((N, 2 * DIM), F32),
                   jax.ShapeDtypeStruct((N, 1), F32)],
    )(s, cnt, h, wtk('conv3_Wl'), b2k('conv3_bl'), wtk('conv3_Wr'),
      wt('lin6'), b2('lin6'), wt('gmt_lin1'), b2('gmt_lin1'),
      wt('mab1_k'), wt('mab1_v'))

    sk = _segsum(hkv[:, :DIM], src, dst)
    sv = _segsum(hkv[:, DIM:], src, dst)

    den, num = pl.pallas_call(
        _pool1_body, grid=(GRID,),
        in_specs=[_seg, _seg, _row2, _col, _col,
                  pl.BlockSpec((NS, DIM), lambda i: (0, 0)),
                  _mat, _bia, _bia, _bia],
        out_specs=[pl.BlockSpec((NG, NH * NS), lambda i: (0, 0)),
                   pl.BlockSpec((NS, NG, DIM), lambda i: (0, 0, 0))],
        out_shape=[jax.ShapeDtypeStruct((NG, NH * NS), F32),
                   jax.ShapeDtypeStruct((NS, NG, DIM), F32)],
    )(sk, sv, hkv, dinv, batch2, p['S1'].reshape(NS, DIM),
      wt('mab1_fcq'), b2('mab1_fcq'), b2('mab1_k'), b2('mab1_v'))

    full = lambda shp: pl.BlockSpec(shp, lambda: tuple(0 for _ in shp))
    args = [num, den, p['S1'].reshape(NS, DIM), p['S3'].reshape(1, DIM),
            wt('mab1_fcq'), b2('mab1_fcq'), wt('mab1_fco'), b2('mab1_fco'),
            wt('mab2_fcq'), b2('mab2_fcq'), wt('mab2_k'), b2('mab2_k'),
            wt('mab2_v'), b2('mab2_v'), wt('mab2_fco'), b2('mab2_fco'),
            wt('mab3_fcq'), b2('mab3_fcq'), wt('mab3_k'), b2('mab3_k'),
            wt('mab3_v'), b2('mab3_v'), wt('mab3_fco'), b2('mab3_fco'),
            p['gmt_lin2_W'].T, p['gmt_lin2_b'].reshape(1, 1)]
    res = pl.pallas_call(
        _pooltail_body,
        in_specs=[full(a.shape) for a in args],
        out_specs=full((NG, 1)),
        out_shape=jax.ShapeDtypeStruct((NG, 1), F32),
    )(*args)
    return res.reshape(-1)
